# baseline (speedup 1.0000x reference)
import functools

import jax
import jax.numpy as jnp
from jax import lax
from jax.experimental import pallas as pl
from jax.experimental.pallas import tpu as pltpu
from jax.experimental.pallas import tpu_sc as plsc

N_ROWS = 100000
D = 128
NSEG = 64
GARBAGE = NSEG
ACC_ROWS = 72
CHUNK = 128
NC, NS = 2, 16
NW = NC * NS
NCHUNKS = -(-N_ROWS // CHUNK)
LAST = NCHUNKS - 1
OVERLAP = NCHUNKS * CHUNK - N_ROWS

SC_CHUNKS = 391
SC_VCHUNKS = SC_CHUNKS + 1
TC_BLOCKS = (LAST) - SC_CHUNKS
MAX_ITERS = -(-SC_VCHUNKS // NW)
CHUNKS_PER_W, EXTRA_W = divmod(SC_VCHUNKS, NW)
HIST_BLK = 8
R_TC = -(-N_ROWS // (HIST_BLK * D)) * HIST_BLK


def _sc_segment_sums(node_feat, ids_aug, zrow):
    mesh = plsc.VectorSubcoreMesh(
        core_axis_name="c", subcore_axis_name="s",
        num_cores=NC, num_subcores=NS)

    @functools.partial(
        pl.kernel,
        out_type=jax.ShapeDtypeStruct((NC, ACC_ROWS, D), jnp.float32),
        mesh=mesh,
        scratch_types=[
            pltpu.VMEM((2, CHUNK, D), jnp.float32),
            pltpu.VMEM((2, CHUNK), jnp.int32),
            pltpu.VMEM((ACC_ROWS, D), jnp.float32),
            pltpu.VMEM_SHARED((ACC_ROWS, D), jnp.float32),
            pltpu.SemaphoreType.DMA,
            pltpu.SemaphoreType.DMA,
        ],
    )
    def seg_sum(feat_hbm, ids_hbm, zrow_hbm, sums_hbm,
                fbuf, ibuf, zbuf, acc_sh, sem0, sem1):
        ci = lax.axis_index("c")
        si = lax.axis_index("s")
        wid = si * NC + ci
        start = wid * CHUNKS_PER_W + jnp.minimum(wid, EXTRA_W)
        n_w = CHUNKS_PER_W + jnp.where(wid < EXTRA_W, 1, 0)
        sems = (sem0, sem1)

        @pl.when(si == 0)
        def _():
            pltpu.sync_copy(zrow_hbm, zbuf)
            pltpu.sync_copy(zbuf, acc_sh)
        plsc.subcore_barrier()

        def issue(i, b):
            v = start + i
            c = jnp.where(v >= SC_CHUNKS, LAST, v)
            feat_base = pl.multiple_of(
                jnp.where(c == LAST, N_ROWS - CHUNK, c * CHUNK), 8)
            idx_base = pl.multiple_of(
                jnp.where(c == LAST, N_ROWS, c * CHUNK), 8)
            pltpu.async_copy(
                feat_hbm.at[pl.ds(feat_base, CHUNK)], fbuf.at[b], sems[b])
            pltpu.async_copy(
                ids_hbm.at[pl.ds(idx_base, CHUNK)], ibuf.at[b], sems[b])

        def wait_load(b):
            pltpu.make_async_copy(
                feat_hbm.at[pl.ds(0, CHUNK)], fbuf.at[b], sems[b]).wait()
            pltpu.make_async_copy(
                ids_hbm.at[pl.ds(0, CHUNK)], ibuf.at[b], sems[b]).wait()

        @pl.when(0 < n_w)
        def _():
            issue(0, 0)

        @pl.when(1 < n_w)
        def _():
            issue(1, 1)

        for i in range(MAX_ITERS):
            b = i & 1

            @pl.when(i < n_w)
            def _(i=i, b=b):
                wait_load(b)
                pltpu.sync_copy(fbuf.at[b], acc_sh.at[ibuf.at[b]], add=True)

                @pl.when(i + 2 < n_w)
                def _():
                    issue(i + 2, b)

        plsc.subcore_barrier()

        @pl.when(si == 0)
        def _():
            pltpu.sync_copy(acc_sh, zbuf)
            pltpu.sync_copy(zbuf, sums_hbm.at[ci])

    return seg_sum(node_feat, ids_aug, zrow)


def _hist_body(i_ref, o_ref):
    r = pl.program_id(0)

    @pl.when(r == 0)
    def _():
        o_ref[...] = jnp.zeros((NSEG, D), jnp.float32)

    seg = lax.broadcasted_iota(jnp.int32, (NSEG, D), 0)
    acc = o_ref[...]
    for j in range(HIST_BLK):
        row = i_ref[j:j + 1, :]
        acc = acc + (jnp.broadcast_to(row, (NSEG, D)) == seg).astype(jnp.float32)
    o_ref[...] = acc


def _tc_histogram(ids_2d):
    return pl.pallas_call(
        _hist_body,
        grid=(R_TC // HIST_BLK,),
        in_specs=[pl.BlockSpec((HIST_BLK, D), lambda r: (r, 0))],
        out_specs=pl.BlockSpec((NSEG, D), lambda r: (0, 0)),
        out_shape=jax.ShapeDtypeStruct((NSEG, D), jnp.float32),
    )(ids_2d)


def _matsum_body(i_ref, f_ref, o_ref):
    r = pl.program_id(0)

    @pl.when(r == 0)
    def _():
        o_ref[...] = jnp.zeros((NSEG, D), jnp.float32)

    seg = lax.broadcasted_iota(jnp.int32, (NSEG, CHUNK), 0)
    onehot = (jnp.broadcast_to(i_ref[0], (NSEG, CHUNK)) == seg).astype(jnp.float32)
    o_ref[...] += jax.lax.dot_general(
        onehot, f_ref[...], (((1,), (0,)), ((), ())),
        preferred_element_type=jnp.float32)


def _tc_matsum(ids_3d, node_feat):
    return pl.pallas_call(
        _matsum_body,
        grid=(TC_BLOCKS,),
        in_specs=[
            pl.BlockSpec((1, 1, CHUNK), lambda r: (r, 0, 0)),
            pl.BlockSpec((CHUNK, D), lambda r: (SC_CHUNKS + r, 0)),
        ],
        out_specs=pl.BlockSpec((NSEG, D), lambda r: (0, 0)),
        out_shape=jax.ShapeDtypeStruct((NSEG, D), jnp.float32),
    )(ids_3d, node_feat)


def _combine_body(s_ref, t_ref, h_ref, o_ref):
    s = s_ref[0, :NSEG, :] + s_ref[1, :NSEG, :] + t_ref[...]
    cnt = jnp.sum(h_ref[...], axis=1, keepdims=True)
    o_ref[...] = s / cnt


def _combine(sums, tsum, hist):
    return pl.pallas_call(
        _combine_body,
        out_shape=jax.ShapeDtypeStruct((NSEG, D), jnp.float32),
    )(sums, tsum, hist)


@jax.jit
def kernel(node_feat, segment_ids):
    ids32 = segment_ids.astype(jnp.int32)
    idx_last = jnp.concatenate(
        [jnp.full((OVERLAP,), GARBAGE, jnp.int32),
         ids32[N_ROWS - CHUNK + OVERLAP:]])
    ids_aug = jnp.concatenate([ids32, idx_last])
    ids_2d = jnp.concatenate(
        [ids32, jnp.full((R_TC * D - N_ROWS,), GARBAGE, jnp.int32)]
    ).reshape(R_TC, D)
    ids_3d = lax.dynamic_slice(
        ids32, (SC_CHUNKS * CHUNK,), (TC_BLOCKS * CHUNK,)
    ).reshape(TC_BLOCKS, 1, CHUNK)
    zrow = jnp.zeros((ACC_ROWS, D), jnp.float32)
    hist = _tc_histogram(ids_2d)
    tsum = _tc_matsum(ids_3d, node_feat)
    sums = _sc_segment_sums(node_feat, ids_aug, zrow)
    return _combine(sums, tsum, hist)


# speedup vs baseline: 2.1065x; 2.1065x over previous
import functools

import jax
import jax.numpy as jnp
from jax import lax
from jax.experimental import pallas as pl
from jax.experimental.pallas import tpu as pltpu
from jax.experimental.pallas import tpu_sc as plsc

N_ROWS = 100000
D = 128
NSEG = 64
GARBAGE = NSEG
ACC_ROWS = 72
CHUNK = 128
NC, NS = 2, 16
NW = NC * NS
NCHUNKS = -(-N_ROWS // CHUNK)
LAST = NCHUNKS - 1
OVERLAP = NCHUNKS * CHUNK - N_ROWS

TC_SUPER = 4
TC_SBLOCKS = 97
TC_BLOCKS = TC_SBLOCKS * TC_SUPER
SC_CHUNKS = 392
SC_VCHUNKS = SC_CHUNKS + 2
NSLAB = 2
MAX_ITERS = -(-SC_VCHUNKS // NW)
CHUNKS_PER_W, EXTRA_W = divmod(SC_VCHUNKS, NW)
HIST_BLK = 8
R_TC = -(-N_ROWS // (HIST_BLK * D)) * HIST_BLK


def _sc_segment_sums(node_feat, ids_aug, zrow):
    mesh = plsc.VectorSubcoreMesh(
        core_axis_name="c", subcore_axis_name="s",
        num_cores=NC, num_subcores=NS)

    @functools.partial(
        pl.kernel,
        out_type=jax.ShapeDtypeStruct((NC, ACC_ROWS, D), jnp.float32),
        mesh=mesh,
        scratch_types=[
            pltpu.VMEM((2, CHUNK, D), jnp.float32),
            pltpu.VMEM((2, CHUNK), jnp.int32),
            pltpu.VMEM((ACC_ROWS, D), jnp.float32),
            pltpu.VMEM_SHARED((ACC_ROWS, D), jnp.float32),
            pltpu.SemaphoreType.DMA,
            pltpu.SemaphoreType.DMA,
        ],
    )
    def seg_sum(feat_hbm, ids_hbm, zrow_hbm, sums_hbm,
                fbuf, ibuf, zbuf, acc_sh, sem0, sem1):
        ci = lax.axis_index("c")
        si = lax.axis_index("s")
        wid = si * NC + ci
        start = wid * CHUNKS_PER_W + jnp.minimum(wid, EXTRA_W)
        n_w = CHUNKS_PER_W + jnp.where(wid < EXTRA_W, 1, 0)
        sems = (sem0, sem1)

        @pl.when(si == 0)
        def _():
            pltpu.sync_copy(zrow_hbm, zbuf)
            pltpu.sync_copy(zbuf, acc_sh)
        plsc.subcore_barrier()

        def issue(i, b):
            v = start + i
            c = jnp.where(v >= SC_CHUNKS, v + TC_BLOCKS, v)
            feat_base = pl.multiple_of(
                jnp.where(c == LAST, N_ROWS - CHUNK, c * CHUNK), 8)
            idx_base = pl.multiple_of(
                jnp.where(c == LAST, N_ROWS, c * CHUNK), 8)
            pltpu.async_copy(
                feat_hbm.at[pl.ds(feat_base, CHUNK)], fbuf.at[b], sems[b])
            pltpu.async_copy(
                ids_hbm.at[pl.ds(idx_base, CHUNK)], ibuf.at[b], sems[b])

        def wait_load(b):
            pltpu.make_async_copy(
                feat_hbm.at[pl.ds(0, CHUNK)], fbuf.at[b], sems[b]).wait()
            pltpu.make_async_copy(
                ids_hbm.at[pl.ds(0, CHUNK)], ibuf.at[b], sems[b]).wait()

        @pl.when(0 < n_w)
        def _():
            issue(0, 0)

        @pl.when(1 < n_w)
        def _():
            issue(1, 1)

        for i in range(MAX_ITERS):
            b = i & 1

            @pl.when(i < n_w)
            def _(i=i, b=b):
                wait_load(b)
                pltpu.sync_copy(fbuf.at[b], acc_sh.at[ibuf.at[b]], add=True)

                @pl.when(i + 2 < n_w)
                def _():
                    issue(i + 2, b)

        plsc.subcore_barrier()

        @pl.when(si == 0)
        def _():
            pltpu.sync_copy(acc_sh, zbuf)
            pltpu.sync_copy(zbuf, sums_hbm.at[ci])

    return seg_sum(node_feat, ids_aug, zrow)


def _hist_body(i_ref, o_ref):
    r = pl.program_id(0)

    @pl.when(r == 0)
    def _():
        o_ref[...] = jnp.zeros((NSEG, D), jnp.float32)

    seg = lax.broadcasted_iota(jnp.int32, (NSEG, D), 0)
    acc = o_ref[...]
    for j in range(HIST_BLK):
        row = i_ref[j:j + 1, :]
        acc = acc + (jnp.broadcast_to(row, (NSEG, D)) == seg).astype(jnp.float32)
    o_ref[...] = acc


def _tc_histogram(ids_2d):
    return pl.pallas_call(
        _hist_body,
        grid=(R_TC // HIST_BLK,),
        in_specs=[pl.BlockSpec((HIST_BLK, D), lambda r: (r, 0))],
        out_specs=pl.BlockSpec((NSEG, D), lambda r: (0, 0)),
        out_shape=jax.ShapeDtypeStruct((NSEG, D), jnp.float32),
    )(ids_2d)


def _matsum_body(i_ref, f_ref, o_ref):
    r = pl.program_id(0)

    @pl.when(r < NSLAB)
    def _():
        o_ref[...] = jnp.zeros((1, NSEG, D), jnp.float32)

    seg = lax.broadcasted_iota(jnp.int32, (NSEG, CHUNK), 0)
    onehot = jnp.concatenate(
        [(jnp.broadcast_to(i_ref[0, j:j + 1, :], (NSEG, CHUNK)) == seg)
         .astype(jnp.bfloat16) for j in range(TC_SUPER)], axis=1)
    feat = f_ref[...].astype(jnp.bfloat16)
    o_ref[0] += jax.lax.dot_general(
        onehot, feat, (((1,), (0,)), ((), ())),
        preferred_element_type=jnp.float32)


def _tc_matsum(ids_3d, node_feat):
    part = pl.pallas_call(
        _matsum_body,
        grid=(TC_SBLOCKS,),
        in_specs=[
            pl.BlockSpec((1, TC_SUPER, CHUNK), lambda r: (r, 0, 0)),
            pl.BlockSpec((TC_SUPER * CHUNK, D),
                         lambda r: (SC_CHUNKS // TC_SUPER + r, 0)),
        ],
        out_specs=pl.BlockSpec((1, NSEG, D), lambda r: (r % NSLAB, 0, 0)),
        out_shape=jax.ShapeDtypeStruct((NSLAB, NSEG, D), jnp.float32),
    )(ids_3d, node_feat)
    return part


def _combine_body(s_ref, t_ref, h_ref, o_ref):
    s = (s_ref[0, :NSEG, :] + s_ref[1, :NSEG, :]
         + t_ref[0, :, :] + t_ref[1, :, :])
    cnt = jnp.sum(h_ref[...], axis=1, keepdims=True)
    o_ref[...] = s / cnt


def _combine(sums, tsum, hist):
    return pl.pallas_call(
        _combine_body,
        out_shape=jax.ShapeDtypeStruct((NSEG, D), jnp.float32),
    )(sums, tsum, hist)


@jax.jit
def kernel(node_feat, segment_ids):
    ids32 = segment_ids.astype(jnp.int32)
    idx_last = jnp.concatenate(
        [jnp.full((OVERLAP,), GARBAGE, jnp.int32),
         ids32[N_ROWS - CHUNK + OVERLAP:]])
    ids_aug = jnp.concatenate([ids32, idx_last])
    ids_2d = jnp.concatenate(
        [ids32, jnp.full((R_TC * D - N_ROWS,), GARBAGE, jnp.int32)]
    ).reshape(R_TC, D)
    ids_3d = lax.dynamic_slice(
        ids32, (SC_CHUNKS * CHUNK,), (TC_BLOCKS * CHUNK,)
    ).reshape(TC_SBLOCKS, TC_SUPER, CHUNK)
    zrow = jnp.zeros((ACC_ROWS, D), jnp.float32)
    hist = _tc_histogram(ids_2d)
    tsum = _tc_matsum(ids_3d, node_feat)
    sums = _sc_segment_sums(node_feat, ids_aug, zrow)
    return _combine(sums, tsum, hist)


# speedup vs baseline: 2.3738x; 1.1269x over previous
import functools

import jax
import jax.numpy as jnp
from jax import lax
from jax.experimental import pallas as pl
from jax.experimental.pallas import tpu as pltpu
from jax.experimental.pallas import tpu_sc as plsc

N_ROWS = 100000
D = 128
NSEG = 64
GARBAGE = NSEG
ACC_ROWS = 72
CHUNK = 128
NC, NS = 2, 16
NW = NC * NS
NCHUNKS = -(-N_ROWS // CHUNK)
LAST = NCHUNKS - 1
OVERLAP = NCHUNKS * CHUNK - N_ROWS
MAX_ITERS = -(-NCHUNKS // NW)
CHUNKS_PER_W, EXTRA_W = divmod(NCHUNKS, NW)
HIST_BLK = 8
R_TC = -(-N_ROWS // (HIST_BLK * D)) * HIST_BLK


def _sc_segment_sums(node_feat, ids_aug, zrow):
    mesh = plsc.VectorSubcoreMesh(
        core_axis_name="c", subcore_axis_name="s",
        num_cores=NC, num_subcores=NS)

    @functools.partial(
        pl.kernel,
        out_type=jax.ShapeDtypeStruct((NC, ACC_ROWS, D), jnp.float32),
        mesh=mesh,
        scratch_types=[
            pltpu.VMEM((2, CHUNK, D), jnp.float32),
            pltpu.VMEM((2, CHUNK), jnp.int32),
            pltpu.VMEM((ACC_ROWS, D), jnp.float32),
            pltpu.VMEM_SHARED((ACC_ROWS, D), jnp.float32),
            pltpu.SemaphoreType.DMA,
            pltpu.SemaphoreType.DMA,
        ],
    )
    def seg_sum(feat_hbm, ids_hbm, zrow_hbm, sums_hbm,
                fbuf, ibuf, zbuf, acc_sh, sem0, sem1):
        ci = lax.axis_index("c")
        si = lax.axis_index("s")
        wid = si * NC + ci
        start = wid * CHUNKS_PER_W + jnp.minimum(wid, EXTRA_W)
        n_w = CHUNKS_PER_W + jnp.where(wid < EXTRA_W, 1, 0)
        sems = (sem0, sem1)

        @pl.when(si == 0)
        def _():
            pltpu.sync_copy(zrow_hbm, zbuf)
            pltpu.sync_copy(zbuf, acc_sh)
        plsc.subcore_barrier()

        def issue(i, b):
            c = start + i
            feat_base = pl.multiple_of(
                jnp.where(c == LAST, N_ROWS - CHUNK, c * CHUNK), 8)
            idx_base = pl.multiple_of(
                jnp.where(c == LAST, N_ROWS, c * CHUNK), 8)
            pltpu.async_copy(
                feat_hbm.at[pl.ds(feat_base, CHUNK)], fbuf.at[b], sems[b])
            pltpu.async_copy(
                ids_hbm.at[pl.ds(idx_base, CHUNK)], ibuf.at[b], sems[b])

        def wait_load(b):
            pltpu.make_async_copy(
                feat_hbm.at[pl.ds(0, CHUNK)], fbuf.at[b], sems[b]).wait()
            pltpu.make_async_copy(
                ids_hbm.at[pl.ds(0, CHUNK)], ibuf.at[b], sems[b]).wait()

        @pl.when(0 < n_w)
        def _():
            issue(0, 0)

        @pl.when(1 < n_w)
        def _():
            issue(1, 1)

        for i in range(MAX_ITERS):
            b = i & 1

            @pl.when(i < n_w)
            def _(i=i, b=b):
                wait_load(b)
                pltpu.sync_copy(fbuf.at[b], acc_sh.at[ibuf.at[b]], add=True)

                @pl.when(i + 2 < n_w)
                def _():
                    issue(i + 2, b)

        plsc.subcore_barrier()

        @pl.when(si == 0)
        def _():
            pltpu.sync_copy(acc_sh, zbuf)
            pltpu.sync_copy(zbuf, sums_hbm.at[ci])

    return seg_sum(node_feat, ids_aug, zrow)


def _hist_combine_body(i_ref, s_ref, h_ref, o_ref):
    r = pl.program_id(0)

    @pl.when(r == 0)
    def _():
        h_ref[...] = jnp.zeros((NSEG, D), jnp.float32)

    seg = lax.broadcasted_iota(jnp.int32, (NSEG, D), 0)
    acc = h_ref[...]
    for j in range(HIST_BLK):
        row = i_ref[j:j + 1, :]
        acc = acc + (jnp.broadcast_to(row, (NSEG, D)) == seg).astype(jnp.float32)
    h_ref[...] = acc

    @pl.when(r == R_TC // HIST_BLK - 1)
    def _():
        s = s_ref[0, :NSEG, :] + s_ref[1, :NSEG, :]
        cnt = jnp.sum(acc, axis=1, keepdims=True)
        o_ref[...] = s / cnt


def _hist_combine(ids_2d, sums):
    out, _ = pl.pallas_call(
        _hist_combine_body,
        grid=(R_TC // HIST_BLK,),
        in_specs=[
            pl.BlockSpec((HIST_BLK, D), lambda r: (r, 0)),
            pl.BlockSpec((NC, ACC_ROWS, D), lambda r: (0, 0, 0)),
        ],
        out_specs=[
            pl.BlockSpec((NSEG, D), lambda r: (0, 0)),
            pl.BlockSpec((NSEG, D), lambda r: (0, 0)),
        ],
        out_shape=[
            jax.ShapeDtypeStruct((NSEG, D), jnp.float32),
            jax.ShapeDtypeStruct((NSEG, D), jnp.float32),
        ],
    )(ids_2d, sums)
    return out


@jax.jit
def kernel(node_feat, segment_ids):
    ids32 = segment_ids.astype(jnp.int32)
    idx_last = jnp.concatenate(
        [jnp.full((OVERLAP,), GARBAGE, jnp.int32),
         ids32[N_ROWS - CHUNK + OVERLAP:]])
    ids_aug = jnp.concatenate([ids32, idx_last])
    ids_2d = jnp.concatenate(
        [ids32, jnp.full((R_TC * D - N_ROWS,), GARBAGE, jnp.int32)]
    ).reshape(R_TC, D)
    zrow = jnp.zeros((ACC_ROWS, D), jnp.float32)
    sums = _sc_segment_sums(node_feat, ids_aug, zrow)
    return _hist_combine(ids_2d, sums)


# speedup vs baseline: 3.5540x; 1.4971x over previous
import functools

import jax
import jax.numpy as jnp
from jax import lax
from jax.experimental import pallas as pl
from jax.experimental.pallas import tpu as pltpu
from jax.experimental.pallas import tpu_sc as plsc

N_ROWS = 100000
D = 128
NSEG = 64
GARBAGE = NSEG
ACC_ROWS = 72
CHUNK = 128
NC, NS = 2, 16
NW = NC * NS
NCHUNKS = -(-N_ROWS // CHUNK)
LAST = NCHUNKS - 1
OVERLAP = NCHUNKS * CHUNK - N_ROWS
MAX_ITERS = -(-NCHUNKS // NW)
CHUNKS_PER_W, EXTRA_W = divmod(NCHUNKS, NW)
HIST_BLK = 8
R_TC = -(-N_ROWS // (HIST_BLK * D)) * HIST_BLK


def _sc_segment_sums(node_feat, ids_aug, zrow):
    mesh = plsc.VectorSubcoreMesh(
        core_axis_name="c", subcore_axis_name="s",
        num_cores=NC, num_subcores=NS)

    @functools.partial(
        pl.kernel,
        out_type=jax.ShapeDtypeStruct((NC, ACC_ROWS, D), jnp.float32),
        mesh=mesh,
        scratch_types=[
            pltpu.VMEM((2, CHUNK, D), jnp.float32),
            pltpu.VMEM((2, CHUNK), jnp.int32),
            pltpu.VMEM((ACC_ROWS, D), jnp.float32),
            pltpu.VMEM_SHARED((ACC_ROWS, D), jnp.float32),
            pltpu.SemaphoreType.DMA,
            pltpu.SemaphoreType.DMA,
        ],
    )
    def seg_sum(feat_hbm, ids_hbm, zrow_hbm, sums_hbm,
                fbuf, ibuf, zbuf, acc_sh, sem0, sem1):
        ci = lax.axis_index("c")
        si = lax.axis_index("s")
        wid = si * NC + ci
        start = wid * CHUNKS_PER_W + jnp.minimum(wid, EXTRA_W)
        n_w = CHUNKS_PER_W + jnp.where(wid < EXTRA_W, 1, 0)
        sems = (sem0, sem1)

        @pl.when(si == 0)
        def _():
            pltpu.sync_copy(zrow_hbm, zbuf)
            pltpu.sync_copy(zbuf, acc_sh)
        plsc.subcore_barrier()

        def issue(i, b):
            c = start + i
            feat_base = pl.multiple_of(
                jnp.where(c == LAST, N_ROWS - CHUNK, c * CHUNK), 8)
            idx_base = pl.multiple_of(
                jnp.where(c == LAST, N_ROWS, c * CHUNK), 8)
            pltpu.async_copy(
                feat_hbm.at[pl.ds(feat_base, CHUNK)], fbuf.at[b], sems[b])
            pltpu.async_copy(
                ids_hbm.at[pl.ds(idx_base, CHUNK)], ibuf.at[b], sems[b])

        def wait_load(b):
            pltpu.make_async_copy(
                feat_hbm.at[pl.ds(0, CHUNK)], fbuf.at[b], sems[b]).wait()
            pltpu.make_async_copy(
                ids_hbm.at[pl.ds(0, CHUNK)], ibuf.at[b], sems[b]).wait()

        @pl.when(0 < n_w)
        def _():
            issue(0, 0)

        @pl.when(1 < n_w)
        def _():
            issue(1, 1)

        for i in range(MAX_ITERS):
            b = i & 1

            @pl.when(i < n_w)
            def _(i=i, b=b):
                wait_load(b)
                pltpu.sync_copy(fbuf.at[b], acc_sh.at[ibuf.at[b]], add=True)

                @pl.when(i + 2 < n_w)
                def _():
                    issue(i + 2, b)

        plsc.subcore_barrier()

        @pl.when(si == 0)
        def _():
            pltpu.sync_copy(acc_sh, zbuf)
            pltpu.sync_copy(zbuf, sums_hbm.at[ci])

    return seg_sum(node_feat, ids_aug, zrow)


def _hist_body(i_ref, o_ref):
    r = pl.program_id(0)

    @pl.when(r == 0)
    def _():
        o_ref[...] = jnp.zeros((NSEG, D), jnp.float32)

    seg = lax.broadcasted_iota(jnp.int32, (NSEG, D), 0)
    acc = o_ref[...]
    for j in range(HIST_BLK):
        row = i_ref[j:j + 1, :]
        acc = acc + (jnp.broadcast_to(row, (NSEG, D)) == seg).astype(jnp.float32)
    o_ref[...] = acc


def _tc_histogram(ids_2d):
    return pl.pallas_call(
        _hist_body,
        grid=(R_TC // HIST_BLK,),
        in_specs=[pl.BlockSpec((HIST_BLK, D), lambda r: (r, 0))],
        out_specs=pl.BlockSpec((NSEG, D), lambda r: (0, 0)),
        out_shape=jax.ShapeDtypeStruct((NSEG, D), jnp.float32),
    )(ids_2d)


def _combine_body(s_ref, h_ref, o_ref):
    s = s_ref[0, :NSEG, :] + s_ref[1, :NSEG, :]
    cnt = jnp.sum(h_ref[...], axis=1, keepdims=True)
    o_ref[...] = s / cnt


def _combine(sums, hist):
    return pl.pallas_call(
        _combine_body,
        out_shape=jax.ShapeDtypeStruct((NSEG, D), jnp.float32),
    )(sums, hist)


@jax.jit
def kernel(node_feat, segment_ids):
    ids32 = segment_ids.astype(jnp.int32)
    idx_last = jnp.concatenate(
        [jnp.full((OVERLAP,), GARBAGE, jnp.int32),
         ids32[N_ROWS - CHUNK + OVERLAP:]])
    ids_aug = jnp.concatenate([ids32, idx_last])
    ids_2d = jnp.concatenate(
        [ids32, jnp.full((R_TC * D - N_ROWS,), GARBAGE, jnp.int32)]
    ).reshape(R_TC, D)
    zrow = jnp.zeros((ACC_ROWS, D), jnp.float32)
    hist = _tc_histogram(ids_2d)
    sums = _sc_segment_sums(node_feat, ids_aug, zrow)
    return _combine(sums, hist)


# speedup vs baseline: 3.5881x; 1.0096x over previous
import functools

import jax
import jax.numpy as jnp
from jax import lax
from jax.experimental import pallas as pl
from jax.experimental.pallas import tpu as pltpu
from jax.experimental.pallas import tpu_sc as plsc

N_ROWS = 100000
D = 128
NSEG = 64
GARBAGE = NSEG
ACC_ROWS = 72
CHUNK = 128
NC, NS = 2, 16
NW = NC * NS
NCHUNKS = -(-N_ROWS // CHUNK)
LAST = NCHUNKS - 1
OVERLAP = NCHUNKS * CHUNK - N_ROWS
MAX_ITERS = -(-NCHUNKS // NW)
CHUNKS_PER_W, EXTRA_W = divmod(NCHUNKS, NW)
HIST_BLK = 8
R_TC = -(-N_ROWS // (HIST_BLK * D)) * HIST_BLK


def _sc_segment_sums(node_feat, ids_aug, zrow):
    mesh = plsc.VectorSubcoreMesh(
        core_axis_name="c", subcore_axis_name="s",
        num_cores=NC, num_subcores=NS)

    @functools.partial(
        pl.kernel,
        out_type=jax.ShapeDtypeStruct((NC, ACC_ROWS, D), jnp.float32),
        mesh=mesh,
        scratch_types=[
            pltpu.VMEM((3, CHUNK, D), jnp.float32),
            pltpu.VMEM((3, CHUNK), jnp.int32),
            pltpu.VMEM((ACC_ROWS, D), jnp.float32),
            pltpu.VMEM_SHARED((ACC_ROWS, D), jnp.float32),
            pltpu.SemaphoreType.DMA,
            pltpu.SemaphoreType.DMA,
            pltpu.SemaphoreType.DMA,
        ],
    )
    def seg_sum(feat_hbm, ids_hbm, zrow_hbm, sums_hbm,
                fbuf, ibuf, zbuf, acc_sh, sem0, sem1, sem2):
        ci = lax.axis_index("c")
        si = lax.axis_index("s")
        wid = si * NC + ci
        start = wid * CHUNKS_PER_W + jnp.minimum(wid, EXTRA_W)
        n_w = CHUNKS_PER_W + jnp.where(wid < EXTRA_W, 1, 0)
        sems = (sem0, sem1, sem2)

        @pl.when(si == 0)
        def _():
            pltpu.sync_copy(zrow_hbm, zbuf)
            pltpu.sync_copy(zbuf, acc_sh)
        plsc.subcore_barrier()

        def issue(i, b):
            c = start + i
            feat_base = pl.multiple_of(
                jnp.where(c == LAST, N_ROWS - CHUNK, c * CHUNK), 8)
            idx_base = pl.multiple_of(
                jnp.where(c == LAST, N_ROWS, c * CHUNK), 8)
            pltpu.async_copy(
                feat_hbm.at[pl.ds(feat_base, CHUNK)], fbuf.at[b], sems[b])
            pltpu.async_copy(
                ids_hbm.at[pl.ds(idx_base, CHUNK)], ibuf.at[b], sems[b])

        def wait_load(b):
            pltpu.make_async_copy(
                feat_hbm.at[pl.ds(0, CHUNK)], fbuf.at[b], sems[b]).wait()
            pltpu.make_async_copy(
                ids_hbm.at[pl.ds(0, CHUNK)], ibuf.at[b], sems[b]).wait()

        for p in range(3):
            @pl.when(p < n_w)
            def _(p=p):
                issue(p, p)

        for i in range(MAX_ITERS):
            b = i % 3

            @pl.when(i < n_w)
            def _(i=i, b=b):
                wait_load(b)
                pltpu.sync_copy(fbuf.at[b], acc_sh.at[ibuf.at[b]], add=True)

                @pl.when(i + 3 < n_w)
                def _():
                    issue(i + 3, b)

        plsc.subcore_barrier()

        @pl.when(si == 0)
        def _():
            pltpu.sync_copy(acc_sh, zbuf)
            pltpu.sync_copy(zbuf, sums_hbm.at[ci])

    return seg_sum(node_feat, ids_aug, zrow)


def _hist_body(i_ref, o_ref):
    r = pl.program_id(0)

    @pl.when(r == 0)
    def _():
        o_ref[...] = jnp.zeros((NSEG, D), jnp.float32)

    seg = lax.broadcasted_iota(jnp.int32, (NSEG, D), 0)
    acc = o_ref[...]
    for j in range(HIST_BLK):
        row = i_ref[j:j + 1, :]
        acc = acc + (jnp.broadcast_to(row, (NSEG, D)) == seg).astype(jnp.float32)
    o_ref[...] = acc


def _tc_histogram(ids_2d):
    return pl.pallas_call(
        _hist_body,
        grid=(R_TC // HIST_BLK,),
        in_specs=[pl.BlockSpec((HIST_BLK, D), lambda r: (r, 0))],
        out_specs=pl.BlockSpec((NSEG, D), lambda r: (0, 0)),
        out_shape=jax.ShapeDtypeStruct((NSEG, D), jnp.float32),
    )(ids_2d)


def _combine_body(s_ref, h_ref, o_ref):
    s = s_ref[0, :NSEG, :] + s_ref[1, :NSEG, :]
    cnt = jnp.sum(h_ref[...], axis=1, keepdims=True)
    o_ref[...] = s / cnt


def _combine(sums, hist):
    return pl.pallas_call(
        _combine_body,
        out_shape=jax.ShapeDtypeStruct((NSEG, D), jnp.float32),
    )(sums, hist)


@jax.jit
def kernel(node_feat, segment_ids):
    ids32 = segment_ids.astype(jnp.int32)
    idx_last = jnp.concatenate(
        [jnp.full((OVERLAP,), GARBAGE, jnp.int32),
         ids32[N_ROWS - CHUNK + OVERLAP:]])
    ids_aug = jnp.concatenate([ids32, idx_last])
    ids_2d = jnp.concatenate(
        [ids32, jnp.full((R_TC * D - N_ROWS,), GARBAGE, jnp.int32)]
    ).reshape(R_TC, D)
    zrow = jnp.zeros((ACC_ROWS, D), jnp.float32)
    hist = _tc_histogram(ids_2d)
    sums = _sc_segment_sums(node_feat, ids_aug, zrow)
    return _combine(sums, hist)
